# baseline (speedup 1.0000x reference)
import functools

import jax
import jax.numpy as jnp
from jax import lax
from jax.experimental import pallas as pl
from jax.experimental.pallas import tpu as pltpu
from jax.experimental.pallas import tpu_sc as plsc

VOCAB = 1000000
DIM = 64
DIMP = 128
BATCH = 16384
NEG = 20

NC = 2
NS = 16
L = 16
NW = NC * NS
BPW = BATCH // NW
CB = 32
NCHUNK = BPW // CB
NROWS = CB * NEG
GSUB = 128
KH = NEG // 2


def _sc_body(tidx_h, cidx_h, nidx_h, wtab_h, ctab_h, pos_h, negf_h,
             tidx_v, cidx_v, trow_v, crow_v, nbuf, nrowc, nhc,
             w_rows, c_rows, n_rows, pos_buf, neg_flat, sem):
    c = lax.axis_index("c")
    s = lax.axis_index("s")
    wid = s * NC + c
    base = wid * BPW

    pltpu.sync_copy(tidx_h.at[pl.ds(base, BPW)], tidx_v)
    pltpu.sync_copy(cidx_h.at[pl.ds(base, BPW)], cidx_v)

    iota = lax.iota(jnp.int32, L)
    iota_neg = iota * NEG

    def tc_rows_body(m, carry):
        tv = tidx_v[pl.ds(m * L, L)]
        trow_v[pl.ds(m * L, L)] = lax.shift_right_logical(tv, 1)
        cv = cidx_v[pl.ds(m * L, L)]
        crow_v[pl.ds(m * L, L)] = lax.shift_right_logical(cv, 1)
        return carry

    lax.fori_loop(0, BPW // L, tc_rows_body, 0)

    def chunk_body(j, carry):
        jb = j * CB
        pltpu.sync_copy(nidx_h.at[pl.ds((base + jb) * NEG, NROWS)], nbuf)

        def ntrans_body(m, carry2):
            v = nbuf[pl.ds(m * L, L)]
            nrowc[pl.ds(m * L, L)] = lax.shift_right_logical(v, 1)
            nhc[pl.ds(m * L, L)] = lax.shift_left(v & 1, 6)
            return carry2

        lax.fori_loop(0, NROWS // L, ntrans_body, 0)

        cps = [
            pltpu.async_copy(wtab_h.at[trow_v.at[pl.ds(jb, CB)]], w_rows, sem),
            pltpu.async_copy(ctab_h.at[crow_v.at[pl.ds(jb, CB)]], c_rows, sem),
        ]
        for g in range(NROWS // GSUB):
            cps.append(pltpu.async_copy(
                ctab_h.at[nrowc.at[pl.ds(g * GSUB, GSUB)]],
                n_rows.at[pl.ds(g * GSUB, GSUB), :], sem))
        for cp in cps:
            cp.wait()

        for g in range(CB // L):
            rowg = iota + g * L
            wh = lax.shift_left(tidx_v[pl.ds(jb + g * L, L)] & 1, 6)
            ch = lax.shift_left(cidx_v[pl.ds(jb + g * L, L)] & 1, 6)

            for half in range(2):
                ks = range(half * KH, (half + 1) * KH)
                nrow_k = [iota_neg + (g * L * NEG + k) for k in ks]
                nh_k = [plsc.load_gather(nhc, [nrow_k[k - half * KH]])
                        for k in ks]

                def d_body(d, accs):
                    dvec = jnp.broadcast_to(d, (L,)).astype(jnp.int32)
                    wv = plsc.load_gather(w_rows, [rowg, wh + dvec])
                    if half == 0:
                        cv = plsc.load_gather(c_rows, [rowg, ch + dvec])
                        accp = accs[0] + wv * cv
                    new_n = []
                    for k in ks:
                        ki = k - half * KH
                        nv = plsc.load_gather(
                            n_rows, [nrow_k[ki], nh_k[ki] + dvec])
                        new_n.append(accs[1][ki] + wv * nv)
                    return ((accp if half == 0 else accs[0]), tuple(new_n))

                zeros = jnp.zeros((L,), jnp.float32)
                accp, accn = lax.fori_loop(
                    0, DIM, d_body, (zeros, (zeros,) * KH))
                if half == 0:
                    pos_buf[pl.ds(jb + g * L, L)] = accp
                for k in ks:
                    neg_flat[pl.ds(k * BPW + jb + g * L, L)] = \
                        accn[k - half * KH]
        return carry

    lax.fori_loop(0, NCHUNK, chunk_body, 0)

    pltpu.sync_copy(pos_buf, pos_h.at[pl.ds(base, BPW)])
    for k in range(NEG):
        pltpu.sync_copy(neg_flat.at[pl.ds(k * BPW, BPW)],
                        negf_h.at[pl.ds(k * BATCH + base, BPW)])


_sc_scores = functools.partial(
    pl.kernel,
    out_type=(
        jax.ShapeDtypeStruct((BATCH,), jnp.float32),
        jax.ShapeDtypeStruct((NEG * BATCH,), jnp.float32),
    ),
    mesh=plsc.VectorSubcoreMesh(
        core_axis_name="c", subcore_axis_name="s", num_cores=NC,
        num_subcores=NS),
    compiler_params=pltpu.CompilerParams(
        needs_layout_passes=False, use_tc_tiling_on_sc=True),
    scratch_types=[
        pltpu.VMEM((BPW,), jnp.int32),
        pltpu.VMEM((BPW,), jnp.int32),
        pltpu.VMEM((BPW,), jnp.int32),
        pltpu.VMEM((BPW,), jnp.int32),
        pltpu.VMEM((NROWS,), jnp.int32),
        pltpu.VMEM((NROWS,), jnp.int32),
        pltpu.VMEM((NROWS,), jnp.int32),
        pltpu.VMEM((CB, DIMP), jnp.float32),
        pltpu.VMEM((CB, DIMP), jnp.float32),
        pltpu.VMEM((NROWS, DIMP), jnp.float32),
        pltpu.VMEM((BPW,), jnp.float32),
        pltpu.VMEM((NEG * BPW,), jnp.float32),
        pltpu.SemaphoreType.DMA,
    ],
)(_sc_body)


def _loss_body(pos_ref, negt_ref, out_ref):
    p = jnp.clip(pos_ref[...], -10.0, 10.0)
    pos_sum = jnp.sum(jax.nn.log_sigmoid(p))
    n = jnp.clip(negt_ref[...], -10.0, 10.0)
    neg_sum = jnp.sum(jax.nn.log_sigmoid(-n))
    loss = -(pos_sum / BATCH) - (neg_sum / (BATCH * NEG))
    out_ref[...] = jnp.broadcast_to(loss, (1, 1))


_loss = pl.pallas_call(
    _loss_body,
    out_shape=jax.ShapeDtypeStruct((1, 1), jnp.float32),
)


def kernel(target_word, context_word, negative_samples, word_embeddings,
           context_embeddings):
    tidx = target_word.astype(jnp.int32)
    cidx = context_word.astype(jnp.int32)
    nidx = negative_samples.astype(jnp.int32).reshape(-1)
    wtab2 = word_embeddings.reshape(VOCAB // 2, DIMP)
    ctab2 = context_embeddings.reshape(VOCAB // 2, DIMP)
    pos, negf = _sc_scores(tidx, cidx, nidx, wtab2, ctab2)
    out = _loss(pos.reshape(BATCH // 128, 128), negf.reshape(NEG, BATCH))
    return out[0, 0]


# speedup vs baseline: 1.6745x; 1.6745x over previous
import functools

import jax
import jax.numpy as jnp
from jax import lax
from jax.experimental import pallas as pl
from jax.experimental.pallas import tpu as pltpu
from jax.experimental.pallas import tpu_sc as plsc

VOCAB = 1000000
DIM = 64
BATCH = 16384
NEG = 20

NC = 2
NS = 16
L = 16
NW = NC * NS
BPW = BATCH // NW
CB = 32
NCHUNK = BPW // CB
NROWS = CB * NEG


def _sc_body(tidx_h, cidx_h, nidx_h, wtab_h, ctab_h, pos_h, negf_h,
             tidx_v, cidx_v, nbuf, w_rows, c_rows, n_rows,
             pos_buf, neg_flat, scr, sem):
    c = lax.axis_index("c")
    s = lax.axis_index("s")
    wid = s * NC + c
    base = wid * BPW

    pltpu.sync_copy(tidx_h.at[pl.ds(base, BPW)], tidx_v)
    pltpu.sync_copy(cidx_h.at[pl.ds(base, BPW)], cidx_v)

    iota = lax.iota(jnp.int32, L)

    def chunk_body(j, carry):
        jb = j * CB
        pltpu.sync_copy(nidx_h.at[pl.ds((base + jb) * NEG, NROWS)], nbuf)

        def issue_wc(m, carry2):
            tv = tidx_v[pl.ds(jb + m * L, L)]
            cv2 = cidx_v[pl.ds(jb + m * L, L)]
            for l in range(L):
                pltpu.async_copy(wtab_h.at[pl.ds(tv[l], 1), :],
                                 w_rows.at[pl.ds(m * L + l, 1), :], sem)
                pltpu.async_copy(ctab_h.at[pl.ds(cv2[l], 1), :],
                                 c_rows.at[pl.ds(m * L + l, 1), :], sem)
            return carry2

        lax.fori_loop(0, CB // L, issue_wc, 0)

        def issue_n(m, carry2):
            nv = nbuf[pl.ds(m * L, L)]
            for l in range(L):
                pltpu.async_copy(ctab_h.at[pl.ds(nv[l], 1), :],
                                 n_rows.at[pl.ds(m * L + l, 1), :], sem)
            return carry2

        lax.fori_loop(0, NROWS // L, issue_n, 0)

        pltpu.make_async_copy(wtab_h.at[pl.ds(0, CB), :], w_rows, sem).wait()
        pltpu.make_async_copy(ctab_h.at[pl.ds(0, CB), :], c_rows, sem).wait()
        pltpu.make_async_copy(ctab_h.at[pl.ds(0, NROWS), :], n_rows,
                              sem).wait()

        def elem_body(i, carry2):
            jbi = jb + i
            wv = [w_rows[i, pl.ds(c * L, L)] for c in range(DIM // L)]
            cv = [c_rows[i, pl.ds(c * L, L)] for c in range(DIM // L)]
            p = (wv[0] * cv[0] + wv[1] * cv[1]) + (wv[2] * cv[2] + wv[3] * cv[3])
            scr[pl.ds(NEG * L, L)] = plsc.cumsum(p)
            nrow = i * NEG
            for k in range(NEG):
                nv = [n_rows[nrow + k, pl.ds(c * L, L)] for c in range(DIM // L)]
                q = (wv[0] * nv[0] + wv[1] * nv[1]) + (wv[2] * nv[2] + wv[3] * nv[3])
                scr[pl.ds(k * L, L)] = plsc.cumsum(q)
            jbi_v = jnp.broadcast_to(jbi, (L,)).astype(jnp.int32)
            t_lo = plsc.load_gather(scr, [iota * L + (L - 1)])
            plsc.store_scatter(neg_flat, [iota * BPW + jbi_v], t_lo)
            t_hi = plsc.load_gather(scr, [(iota + L) * L + (L - 1)])
            plsc.store_scatter(neg_flat, [(iota + L) * BPW + jbi_v], t_hi,
                               mask=iota < (NEG - L))
            plsc.store_scatter(pos_buf, [jbi_v], t_hi,
                               mask=iota == (NEG - L))
            return carry2

        lax.fori_loop(0, CB, elem_body, 0)
        return carry

    lax.fori_loop(0, NCHUNK, chunk_body, 0)

    pltpu.sync_copy(pos_buf, pos_h.at[pl.ds(base, BPW)])
    for k in range(NEG):
        pltpu.sync_copy(neg_flat.at[pl.ds(k * BPW, BPW)],
                        negf_h.at[pl.ds(k * BATCH + base, BPW)])


_sc_scores = functools.partial(
    pl.kernel,
    out_type=(
        jax.ShapeDtypeStruct((BATCH,), jnp.float32),
        jax.ShapeDtypeStruct((NEG * BATCH,), jnp.float32),
    ),
    mesh=plsc.VectorSubcoreMesh(
        core_axis_name="c", subcore_axis_name="s", num_cores=NC,
        num_subcores=NS),
    compiler_params=pltpu.CompilerParams(
        needs_layout_passes=False, use_tc_tiling_on_sc=True),
    scratch_types=[
        pltpu.VMEM((BPW,), jnp.int32),
        pltpu.VMEM((BPW,), jnp.int32),
        pltpu.VMEM((NROWS,), jnp.int32),
        pltpu.VMEM((CB, DIM), jnp.float32),
        pltpu.VMEM((CB, DIM), jnp.float32),
        pltpu.VMEM((NROWS, DIM), jnp.float32),
        pltpu.VMEM((BPW,), jnp.float32),
        pltpu.VMEM((NEG * BPW,), jnp.float32),
        pltpu.VMEM((2 * NEG * L,), jnp.float32),
        pltpu.SemaphoreType.DMA,
    ],
)(_sc_body)


def _loss_body(pos_ref, negt_ref, out_ref):
    p = jnp.clip(pos_ref[...], -10.0, 10.0)
    pos_sum = jnp.sum(jax.nn.log_sigmoid(p))
    n = jnp.clip(negt_ref[...], -10.0, 10.0)
    neg_sum = jnp.sum(jax.nn.log_sigmoid(-n))
    loss = -(pos_sum / BATCH) - (neg_sum / (BATCH * NEG))
    out_ref[...] = jnp.broadcast_to(loss, (1, 1))


_loss = pl.pallas_call(
    _loss_body,
    out_shape=jax.ShapeDtypeStruct((1, 1), jnp.float32),
)


def kernel(target_word, context_word, negative_samples, word_embeddings,
           context_embeddings):
    tidx = target_word.astype(jnp.int32)
    cidx = context_word.astype(jnp.int32)
    nidx = negative_samples.astype(jnp.int32).reshape(-1)
    pos, negf = _sc_scores(tidx, cidx, nidx, word_embeddings,
                           context_embeddings)
    out = _loss(pos.reshape(BATCH // 128, 128), negf.reshape(NEG, BATCH))
    return out[0, 0]


# speedup vs baseline: 1.7005x; 1.0155x over previous
import functools

import jax
import jax.numpy as jnp
from jax import lax
from jax.experimental import pallas as pl
from jax.experimental.pallas import tpu as pltpu
from jax.experimental.pallas import tpu_sc as plsc

VOCAB = 1000000
DIM = 64
BATCH = 16384
NEG = 20

NC = 2
NS = 16
L = 16
NW = NC * NS
BPW = BATCH // NW
CB = 16
NCHUNK = BPW // CB
NROWS = CB * NEG


def _sc_body(tidx_h, cidx_h, nidx_h, wtab_h, ctab_h, pos_h, negf_h,
             tidx_v, cidx_v, nbuf_a, nbuf_b, w_a, c_a, n_a, w_b, c_b, n_b,
             pos_buf, neg_flat, scr, sem_a, sem_b):
    c = lax.axis_index("c")
    s = lax.axis_index("s")
    wid = s * NC + c
    base = wid * BPW

    pltpu.sync_copy(tidx_h.at[pl.ds(base, BPW)], tidx_v)
    pltpu.sync_copy(cidx_h.at[pl.ds(base, BPW)], cidx_v)

    iota = lax.iota(jnp.int32, L)

    def issue(j, nbuf, w_rows, c_rows, n_rows, sem):
        jb = j * CB
        pltpu.sync_copy(nidx_h.at[pl.ds((base + jb) * NEG, NROWS)], nbuf)
        tv = tidx_v[pl.ds(jb, L)]
        cv2 = cidx_v[pl.ds(jb, L)]
        for l in range(L):
            pltpu.async_copy(wtab_h.at[pl.ds(tv[l], 1), :],
                             w_rows.at[pl.ds(l, 1), :], sem)
            pltpu.async_copy(ctab_h.at[pl.ds(cv2[l], 1), :],
                             c_rows.at[pl.ds(l, 1), :], sem)

        def issue_n(m, carry2):
            nv = nbuf[pl.ds(m * L, L)]
            for l in range(L):
                pltpu.async_copy(ctab_h.at[pl.ds(nv[l], 1), :],
                                 n_rows.at[pl.ds(m * L + l, 1), :], sem)
            return carry2

        lax.fori_loop(0, NROWS // L, issue_n, 0)

    def drain(w_rows, c_rows, n_rows, sem):
        pltpu.make_async_copy(wtab_h.at[pl.ds(0, CB), :], w_rows, sem).wait()
        pltpu.make_async_copy(ctab_h.at[pl.ds(0, CB), :], c_rows, sem).wait()
        pltpu.make_async_copy(ctab_h.at[pl.ds(0, NROWS), :], n_rows,
                              sem).wait()

    def compute(j, w_rows, c_rows, n_rows):
        jb = j * CB

        def elem_body(i, carry2):
            jbi = jb + i
            wv = [w_rows[i, pl.ds(c * L, L)] for c in range(DIM // L)]
            cv = [c_rows[i, pl.ds(c * L, L)] for c in range(DIM // L)]
            p = (wv[0] * cv[0] + wv[1] * cv[1]) + (wv[2] * cv[2] + wv[3] * cv[3])
            scr[pl.ds(NEG * L, L)] = plsc.cumsum(p)
            nrow = i * NEG
            for k in range(NEG):
                nv = [n_rows[nrow + k, pl.ds(c * L, L)] for c in range(DIM // L)]
                q = (wv[0] * nv[0] + wv[1] * nv[1]) + (wv[2] * nv[2] + wv[3] * nv[3])
                scr[pl.ds(k * L, L)] = plsc.cumsum(q)
            jbi_v = jnp.broadcast_to(jbi, (L,)).astype(jnp.int32)
            t_lo = plsc.load_gather(scr, [iota * L + (L - 1)])
            plsc.store_scatter(neg_flat, [iota * BPW + jbi_v], t_lo)
            t_hi = plsc.load_gather(scr, [(iota + L) * L + (L - 1)])
            plsc.store_scatter(neg_flat, [(iota + L) * BPW + jbi_v], t_hi,
                               mask=iota < (NEG - L))
            plsc.store_scatter(pos_buf, [jbi_v], t_hi,
                               mask=iota == (NEG - L))
            return carry2

        lax.fori_loop(0, CB, elem_body, 0)

    issue(0, nbuf_a, w_a, c_a, n_a, sem_a)

    def body(jj, carry):
        a = 2 * jj
        issue(a + 1, nbuf_b, w_b, c_b, n_b, sem_b)
        drain(w_a, c_a, n_a, sem_a)
        compute(a, w_a, c_a, n_a)

        @pl.when(a + 2 < NCHUNK)
        def _():
            issue(a + 2, nbuf_a, w_a, c_a, n_a, sem_a)

        drain(w_b, c_b, n_b, sem_b)
        compute(a + 1, w_b, c_b, n_b)
        return carry

    lax.fori_loop(0, NCHUNK // 2, body, 0)

    pltpu.sync_copy(pos_buf, pos_h.at[pl.ds(base, BPW)])
    for k in range(NEG):
        pltpu.sync_copy(neg_flat.at[pl.ds(k * BPW, BPW)],
                        negf_h.at[pl.ds(k * BATCH + base, BPW)])


_sc_scores = functools.partial(
    pl.kernel,
    out_type=(
        jax.ShapeDtypeStruct((BATCH,), jnp.float32),
        jax.ShapeDtypeStruct((NEG * BATCH,), jnp.float32),
    ),
    mesh=plsc.VectorSubcoreMesh(
        core_axis_name="c", subcore_axis_name="s", num_cores=NC,
        num_subcores=NS),
    compiler_params=pltpu.CompilerParams(
        needs_layout_passes=False, use_tc_tiling_on_sc=True),
    scratch_types=[
        pltpu.VMEM((BPW,), jnp.int32),
        pltpu.VMEM((BPW,), jnp.int32),
        pltpu.VMEM((NROWS,), jnp.int32),
        pltpu.VMEM((NROWS,), jnp.int32),
        pltpu.VMEM((CB, DIM), jnp.float32),
        pltpu.VMEM((CB, DIM), jnp.float32),
        pltpu.VMEM((NROWS, DIM), jnp.float32),
        pltpu.VMEM((CB, DIM), jnp.float32),
        pltpu.VMEM((CB, DIM), jnp.float32),
        pltpu.VMEM((NROWS, DIM), jnp.float32),
        pltpu.VMEM((BPW,), jnp.float32),
        pltpu.VMEM((NEG * BPW,), jnp.float32),
        pltpu.VMEM((2 * NEG * L,), jnp.float32),
        pltpu.SemaphoreType.DMA,
        pltpu.SemaphoreType.DMA,
    ],
)(_sc_body)


def _loss_body(pos_ref, negt_ref, out_ref):
    p = jnp.clip(pos_ref[...], -10.0, 10.0)
    pos_sum = jnp.sum(jax.nn.log_sigmoid(p))
    n = jnp.clip(negt_ref[...], -10.0, 10.0)
    neg_sum = jnp.sum(jax.nn.log_sigmoid(-n))
    loss = -(pos_sum / BATCH) - (neg_sum / (BATCH * NEG))
    out_ref[...] = jnp.broadcast_to(loss, (1, 1))


_loss = pl.pallas_call(
    _loss_body,
    out_shape=jax.ShapeDtypeStruct((1, 1), jnp.float32),
)


def kernel(target_word, context_word, negative_samples, word_embeddings,
           context_embeddings):
    tidx = target_word.astype(jnp.int32)
    cidx = context_word.astype(jnp.int32)
    nidx = negative_samples.astype(jnp.int32).reshape(-1)
    pos, negf = _sc_scores(tidx, cidx, nidx, word_embeddings,
                           context_embeddings)
    out = _loss(pos.reshape(BATCH // 128, 128), negf.reshape(NEG, BATCH))
    return out[0, 0]
